# baseline (speedup 1.0000x reference)
import functools

import jax
import jax.numpy as jnp
from jax import lax
from jax.experimental import pallas as pl
from jax.experimental.pallas import tpu as pltpu
from jax.experimental.pallas import tpu_sc as plsc

N = 10000
K = 32
D = 128
L = 16
DB = D // L
W = D // 2
CHUNK_NODES = 4
ROWS = CHUNK_NODES * K
NCHUNKS = N // CHUNK_NODES
NC = 2
NS = 16
NW = NC * NS
SUP = 8
NSUPER = 10
PAD_CHUNKS = NW * SUP * NSUPER
NPAD = PAD_CHUNKS * CHUNK_NODES
IWC = ROWS * 2
GDEPTH = 4


def _build():
    mesh = plsc.VectorSubcoreMesh(
        core_axis_name="c", subcore_axis_name="s", num_cores=NC, num_subcores=NS
    )

    @functools.partial(
        pl.kernel,
        mesh=mesh,
        out_type=jax.ShapeDtypeStruct((NPAD, D), jnp.float32),
        scratch_types=[
            pltpu.VMEM_SHARED((N, W), jnp.int32),
            pltpu.VMEM((2, SUP, IWC), jnp.int32),
            pltpu.VMEM((GDEPTH, ROWS, W), jnp.int32),
            pltpu.VMEM((ROWS,), jnp.float32),
            pltpu.VMEM((2, CHUNK_NODES, D), jnp.float32),
            pltpu.SemaphoreType.DMA,
            [pltpu.SemaphoreType.DMA] * GDEPTH,
            [pltpu.SemaphoreType.DMA] * 2,
        ],
        compiler_params=pltpu.CompilerParams(
            needs_layout_passes=False, use_tc_tiling_on_sc=False
        ),
    )
    def body(x_hbm, idxw_hbm, out_hbm, x_sp, iw, rows, w_chunk, out_v,
             sem_iw, sem_g, sem_o):
        wid = lax.axis_index("s") * NC + lax.axis_index("c")
        lane = lax.iota(jnp.int32, L)
        hi_mask = jnp.full((L,), -65536, jnp.int32)

        KU = 8

        def reduce_chunk(par, j, c):
            rows_v = rows.at[(j + 0) % GDEPTH]
            po = j % 2
            for b in range(ROWS // L):
                w_chunk[pl.ds(b * L, L)] = plsc.bitcast(
                    iw[par, j, pl.ds(ROWS + b * L, L)], jnp.float32
                )
            pltpu.make_async_copy(
                out_v.at[po], out_hbm.at[pl.ds(0, CHUNK_NODES)], sem_o[po]
            ).wait()

            def n_body(n, carry):
                kb = n * K
                w0 = w_chunk[pl.ds(kb, L)]
                w1 = w_chunk[pl.ds(kb + L, L)]
                t = w0 + w1
                for sh in (8, 4, 2, 1):
                    t = t + t.at[(lane + sh) & (L - 1)].get(
                        mode="promise_in_bounds"
                    )
                inv = jnp.where(t > 0.0, 1.0 / t, 1.0)

                def k_body(i, accs, kb=kb):
                    accs = list(accs)
                    for jj in range(KU):
                        r = kb + i * KU + jj
                        wk = plsc.load_gather(
                            w_chunk, [jnp.full((L,), r, jnp.int32)]
                        )
                        for wb in range(W // L):
                            word = rows_v[r, pl.ds(wb * L, L)]
                            lo = plsc.bitcast(word << 16, jnp.float32)
                            hi = plsc.bitcast(word & hi_mask, jnp.float32)
                            accs[wb] = accs[wb] + wk * lo
                            accs[wb + 4] = accs[wb + 4] + wk * hi
                    return tuple(accs)

                accs = lax.fori_loop(
                    0, K // KU, k_body,
                    tuple(jnp.zeros((L,), jnp.float32) for _ in range(DB)),
                )
                for db in range(DB):
                    out_v[po, n, pl.ds(db * L, L)] = accs[db] * inv
                return carry

            lax.fori_loop(0, CHUNK_NODES, n_body, 0)
            pltpu.async_copy(
                out_v.at[po],
                out_hbm.at[pl.ds(c * CHUNK_NODES, CHUNK_NODES)],
                sem_o[po],
            )

        @pl.when(lax.axis_index("s") == 0)
        def _():
            pltpu.sync_copy(x_hbm, x_sp)

        plsc.subcore_barrier()

        def sup_start(s):
            return (wid + s * NW) * SUP

        def fire_gather(idx_slice, t):
            pltpu.async_copy(x_sp.at[idx_slice], rows.at[t % GDEPTH],
                             sem_g[t % GDEPTH])

        def wait_gather(t):
            pltpu.make_async_copy(
                x_sp.at[iw.at[0, 0, pl.ds(0, ROWS)]], rows.at[t % GDEPTH],
                sem_g[t % GDEPTH]
            ).wait()

        for po in range(2):
            pltpu.async_copy(
                out_v.at[po],
                out_hbm.at[pl.ds((PAD_CHUNKS - 1) * CHUNK_NODES, CHUNK_NODES)],
                sem_o[po],
            )
        pltpu.sync_copy(idxw_hbm.at[pl.ds(sup_start(0), SUP)], iw.at[0])
        for t in range(GDEPTH - 1):
            fire_gather(iw.at[0, t, pl.ds(0, ROWS)], t)

        def s_body(s, carry):
            par = jnp.bitwise_and(s, 1)
            nxt = 1 - par
            nstart = jnp.minimum(sup_start(s + 1), PAD_CHUNKS - SUP)
            pltpu.async_copy(idxw_hbm.at[pl.ds(nstart, SUP)], iw.at[nxt], sem_iw)
            for j in range(SUP):
                c = sup_start(s) + j
                fire = j + GDEPTH - 1
                if fire >= SUP:
                    if fire == SUP:
                        pltpu.make_async_copy(
                            idxw_hbm.at[pl.ds(nstart, SUP)], iw.at[nxt], sem_iw
                        ).wait()
                    nidx = iw.at[nxt, fire - SUP, pl.ds(0, ROWS)]
                else:
                    nidx = iw.at[par, fire, pl.ds(0, ROWS)]
                fire_gather(nidx, fire)
                wait_gather(j)
                reduce_chunk(par, j, c)
            return carry

        lax.fori_loop(0, NSUPER, s_body, 0)
        for t in range(GDEPTH - 1):
            wait_gather(t)
        for po in range(2):
            pltpu.make_async_copy(
                out_v.at[po], out_hbm.at[pl.ds(0, CHUNK_NODES)], sem_o[po]
            ).wait()

    return body


_sc_pool = _build()


def kernel(x, neighbors, weights):
    xb = x.astype(jnp.bfloat16)
    lo = lax.bitcast_convert_type(xb[:, : D // 2], jnp.uint16).astype(jnp.uint32)
    hi = lax.bitcast_convert_type(xb[:, D // 2 :], jnp.uint16).astype(jnp.uint32)
    xp = lax.bitcast_convert_type(lo | (hi << 16), jnp.int32)
    nbr = neighbors.astype(jnp.int32).reshape(NCHUNKS, ROWS)
    wct = lax.bitcast_convert_type(
        weights.astype(jnp.float32), jnp.int32
    ).reshape(NCHUNKS, ROWS)
    idxw = jnp.concatenate([nbr, wct], axis=1)
    idxw = jnp.pad(idxw, ((0, PAD_CHUNKS - NCHUNKS), (0, 0)))
    return _sc_pool(xp, idxw)[:N]


# speedup vs baseline: 1.0839x; 1.0839x over previous
import functools

import jax
import jax.numpy as jnp
from jax import lax
from jax.experimental import pallas as pl
from jax.experimental.pallas import tpu as pltpu
from jax.experimental.pallas import tpu_sc as plsc

N = 10000
K = 32
D = 128
L = 16
DB = D // L
W = D // 2
CHUNK_NODES = 4
ROWS = CHUNK_NODES * K
NCHUNKS = N // CHUNK_NODES
NC = 2
NS = 16
NW = NC * NS
SUP = 8
NSUPER = 10
PAD_CHUNKS = NW * SUP * NSUPER
SUPW = SUP * ROWS
KU = 8


def _build():
    mesh = plsc.VectorSubcoreMesh(
        core_axis_name="c", subcore_axis_name="s", num_cores=NC, num_subcores=NS
    )

    @functools.partial(
        pl.kernel,
        mesh=mesh,
        out_type=jax.ShapeDtypeStruct((N, D), jnp.float32),
        scratch_types=[
            pltpu.VMEM_SHARED((N, W), jnp.int32),
            pltpu.VMEM((2, SUPW), jnp.int32),
            pltpu.VMEM((2, SUPW), jnp.float32),
            pltpu.VMEM((ROWS, W), jnp.int32),
            pltpu.VMEM((ROWS, W), jnp.int32),
            pltpu.VMEM((ROWS,), jnp.float32),
            pltpu.VMEM((CHUNK_NODES, D), jnp.float32),
            pltpu.SemaphoreType.DMA,
            pltpu.SemaphoreType.DMA,
            pltpu.SemaphoreType.DMA,
        ],
        compiler_params=pltpu.CompilerParams(
            needs_layout_passes=False, use_tc_tiling_on_sc=False
        ),
    )
    def body(x_hbm, nbr_hbm, w_hbm, out_hbm, x_sp, nb, wb, rows_a, rows_b,
             w_chunk, out_v, sem_iw, sem_a, sem_b):
        wid = lax.axis_index("s") * NC + lax.axis_index("c")
        lane = lax.iota(jnp.int32, L)
        hi_mask = jnp.full((L,), -65536, jnp.int32)

        def reduce_chunk(par, j, rows_v, c):
            for b in range(ROWS // L):
                w_chunk[pl.ds(b * L, L)] = wb[par, pl.ds(j * ROWS + b * L, L)]

            def n_body(n, carry):
                kb = n * K
                w0 = w_chunk[pl.ds(kb, L)]
                w1 = w_chunk[pl.ds(kb + L, L)]
                t = w0 + w1
                for sh in (8, 4, 2, 1):
                    t = t + t.at[(lane + sh) & (L - 1)].get(
                        mode="promise_in_bounds"
                    )
                inv = jnp.where(t > 0.0, 1.0 / t, 1.0)

                def k_body(i, accs, kb=kb):
                    accs = list(accs)
                    for jj in range(KU):
                        r = kb + i * KU + jj
                        wk = plsc.load_gather(
                            w_chunk, [jnp.full((L,), r, jnp.int32)]
                        )
                        for wb_ in range(W // L):
                            word = rows_v[r, pl.ds(wb_ * L, L)]
                            lo = plsc.bitcast(word << 16, jnp.float32)
                            hi = plsc.bitcast(word & hi_mask, jnp.float32)
                            accs[wb_] = accs[wb_] + wk * lo
                            accs[wb_ + 4] = accs[wb_ + 4] + wk * hi
                    return tuple(accs)

                accs = lax.fori_loop(
                    0, K // KU, k_body,
                    tuple(jnp.zeros((L,), jnp.float32) for _ in range(DB)),
                )
                for db in range(DB):
                    out_v[n, pl.ds(db * L, L)] = accs[db] * inv
                return carry

            lax.fori_loop(0, CHUNK_NODES, n_body, 0)
            pltpu.sync_copy(
                out_v, out_hbm.at[pl.ds(c * CHUNK_NODES, CHUNK_NODES)]
            )

        @pl.when(lax.axis_index("s") == 0)
        def _():
            pltpu.sync_copy(x_hbm, x_sp)

        plsc.subcore_barrier()

        def sup_start(s):
            return (wid + s * NW) * SUP

        pltpu.sync_copy(nbr_hbm.at[pl.ds(sup_start(0) * ROWS, SUPW)], nb.at[0])
        pltpu.sync_copy(w_hbm.at[pl.ds(sup_start(0) * ROWS, SUPW)], wb.at[0])
        pltpu.async_copy(x_sp.at[nb.at[0, pl.ds(0, ROWS)]], rows_a, sem_a)

        def s_body(s, carry):
            par = jnp.bitwise_and(s, 1)
            nxt = 1 - par
            nstart = jnp.minimum(sup_start(s + 1), PAD_CHUNKS - SUP) * ROWS
            pltpu.async_copy(nbr_hbm.at[pl.ds(nstart, SUPW)], nb.at[nxt], sem_iw)
            pltpu.async_copy(w_hbm.at[pl.ds(nstart, SUPW)], wb.at[nxt], sem_iw)
            for j in range(SUP):
                c = sup_start(s) + j
                rv, sv = (rows_a, sem_a) if j % 2 == 0 else (rows_b, sem_b)
                rn, sn = (rows_b, sem_b) if j % 2 == 0 else (rows_a, sem_a)
                if j == SUP - 1:
                    pltpu.make_async_copy(
                        nbr_hbm.at[pl.ds(nstart, SUPW)], nb.at[nxt], sem_iw
                    ).wait()
                    pltpu.make_async_copy(
                        w_hbm.at[pl.ds(nstart, SUPW)], wb.at[nxt], sem_iw
                    ).wait()
                    nidx = nb.at[nxt, pl.ds(0, ROWS)]
                else:
                    nidx = nb.at[par, pl.ds((j + 1) * ROWS, ROWS)]
                pltpu.async_copy(x_sp.at[nidx], rn, sn)
                pltpu.make_async_copy(x_sp.at[nidx], rv, sv).wait()

                @pl.when(c < NCHUNKS)
                def _():
                    reduce_chunk(par, j, rv, c)

            return carry

        lax.fori_loop(0, NSUPER, s_body, 0)
        pltpu.make_async_copy(
            x_sp.at[nb.at[0, pl.ds(0, ROWS)]], rows_a, sem_a
        ).wait()

    return body


_sc_pool = _build()


def kernel(x, neighbors, weights):
    xb = x.astype(jnp.bfloat16)
    lo = lax.bitcast_convert_type(xb[:, : D // 2], jnp.uint16).astype(jnp.uint32)
    hi = lax.bitcast_convert_type(xb[:, D // 2 :], jnp.uint16).astype(jnp.uint32)
    xp = lax.bitcast_convert_type(lo | (hi << 16), jnp.int32)
    pad = PAD_CHUNKS * ROWS - N * K
    nbr = jnp.pad(neighbors.astype(jnp.int32).reshape(-1), (0, pad))
    wf = jnp.pad(weights.astype(jnp.float32).reshape(-1), (0, pad))
    return _sc_pool(xp, nbr, wf)
